# baseline (speedup 1.0000x reference)
import functools

import jax
import jax.numpy as jnp
from jax import lax
from jax.experimental import pallas as pl
from jax.experimental.pallas import tpu as pltpu
from jax.experimental.pallas import tpu_sc as plsc

_NC = 2
_NS = 16
_L = 16
_CH = 1024
_HF = 64
_ZB = 125


def _sc_spmm(Xh, srcp, dstp, wp, T, N, F):
    TT = T // _NC
    Epad = srcp.shape[0]
    ep_tile = Epad // _NS
    nchunk = ep_tile // _CH
    stripe = N // _NS
    nf = _HF // _L

    mesh = plsc.VectorSubcoreMesh(core_axis_name="c", subcore_axis_name="s",
                                  num_cores=_NC, num_subcores=_NS)

    @functools.partial(
        pl.kernel,
        out_type=jax.ShapeDtypeStruct((T, N, F), jnp.float32),
        mesh=mesh,
        compiler_params=pltpu.CompilerParams(use_tc_tiling_on_sc=False),
        scratch_types=[
            pltpu.VMEM((_CH,), jnp.int32),
            pltpu.VMEM((_CH,), jnp.int32),
            pltpu.VMEM((_CH,), jnp.int32),
            pltpu.VMEM((_CH,), jnp.float32),
            pltpu.VMEM((_CH, _HF), jnp.float32),
            pltpu.VMEM((_ZB, _HF), jnp.float32),
            pltpu.VMEM_SHARED((N, _HF), jnp.float32),
        ],
    )
    def spmm(x_hbm, src_hbm, dst_hbm, w_hbm, out_hbm,
             src_buf, idx_buf, dst_buf, w_buf, rows_buf, zero_buf, acc):
        c = lax.axis_index("c")
        s = lax.axis_index("s")
        row0 = s * stripe

        zvec = jnp.zeros((_L,), jnp.float32)

        def zrow_body(r, _):
            for ff in range(nf):
                zero_buf[r, pl.ds(ff * _L, _L)] = zvec
            return 0
        lax.fori_loop(0, _ZB, zrow_body, 0)

        for tt in range(TT):
            t = c * TT + tt
            for half in range(2):
                base_idx = (t * N) * 2 + half

                for z in range(stripe // _ZB):
                    pltpu.sync_copy(
                        zero_buf, acc.at[pl.ds(row0 + z * _ZB, _ZB), :])
                plsc.subcore_barrier()

                def chunk_body(ci, _):
                    e0 = s * ep_tile + ci * _CH
                    pltpu.sync_copy(src_hbm.at[pl.ds(e0, _CH)], src_buf)
                    pltpu.sync_copy(dst_hbm.at[pl.ds(e0, _CH)], dst_buf)
                    pltpu.sync_copy(w_hbm.at[t, pl.ds(e0, _CH)], w_buf)

                    def off_body(g, _):
                        sl = pl.ds(g * _L, _L)
                        idx_buf[sl] = src_buf[sl] * 2 + base_idx
                        return 0
                    lax.fori_loop(0, _CH // _L, off_body, 0)

                    pltpu.sync_copy(x_hbm.at[idx_buf], rows_buf)

                    def scale_body(g, _):
                        w16 = w_buf[pl.ds(g * _L, _L)]
                        base = g * _L
                        for l in range(_L):
                            wl = lax.broadcast(w16[l], (_L,))
                            for ff in range(nf):
                                fsl = pl.ds(ff * _L, _L)
                                rows_buf[base + l, fsl] = (
                                    rows_buf[base + l, fsl] * wl)
                        return 0
                    lax.fori_loop(0, _CH // _L, scale_body, 0)

                    pltpu.sync_copy(rows_buf, acc.at[dst_buf], add=True)
                    return 0
                lax.fori_loop(0, nchunk, chunk_body, 0)
                plsc.subcore_barrier()

                pltpu.sync_copy(
                    acc.at[pl.ds(row0, stripe), :],
                    out_hbm.at[t, pl.ds(row0, stripe),
                               pl.ds(half * _HF, _HF)])
                plsc.subcore_barrier()

    return spmm(Xh, srcp, dstp, wp)


def _bdot(a, b):
    return jnp.dot(a.astype(jnp.bfloat16), b.astype(jnp.bfloat16),
                   preferred_element_type=jnp.float32)


def _lstm_body(ax_ref, w_ref, wg_ref, ug_ref, bg_ref, h0_ref, c0_ref,
               lw_ref, lb_ref, out_ref):
    T, B, H = ax_ref.shape[0], ax_ref.shape[1], wg_ref.shape[0]
    w = w_ref[...]
    h = jnp.broadcast_to(h0_ref[...], (B, H))
    c = jnp.broadcast_to(c0_ref[...], (B, H))
    wg = wg_ref[...]
    ug = ug_ref[...]
    bg = bg_ref[...]
    lw = lw_ref[...]
    lb = lb_ref[0, 0]
    zs = []
    for t in range(T):
        y = jnp.maximum(_bdot(ax_ref[t], w), 0.0)
        g = _bdot(y, wg) + _bdot(h, ug) + bg
        f = jax.nn.sigmoid(g[:, 0:H])
        j = jax.nn.sigmoid(g[:, H:2 * H])
        o = jax.nn.sigmoid(g[:, 2 * H:3 * H])
        ct = jax.nn.sigmoid(g[:, 3 * H:4 * H])
        c = j * ct + f * c
        h = o * jnp.tanh(c)
        zs.append(_bdot(h, lw) + lb)
    out_ref[...] = jnp.concatenate(zs, axis=1)


def _lstm(AX, W, Wg, Ug, bg, h0, c0, lwT, lb):
    T, N, F = AX.shape
    H = W.shape[1]
    B = 2000
    return pl.pallas_call(
        _lstm_body,
        grid=(N // B,),
        in_specs=[
            pl.BlockSpec((T, B, F), lambda i: (0, i, 0)),
            pl.BlockSpec((F, H), lambda i: (0, 0)),
            pl.BlockSpec((H, 4 * H), lambda i: (0, 0)),
            pl.BlockSpec((H, 4 * H), lambda i: (0, 0)),
            pl.BlockSpec((1, 4 * H), lambda i: (0, 0)),
            pl.BlockSpec((1, H), lambda i: (0, 0)),
            pl.BlockSpec((1, H), lambda i: (0, 0)),
            pl.BlockSpec((H, 1), lambda i: (0, 0)),
            pl.BlockSpec((1, 1), lambda i: (0, 0)),
        ],
        out_specs=pl.BlockSpec((B, T), lambda i: (i, 0)),
        out_shape=jax.ShapeDtypeStruct((N, T), jnp.float32),
    )(AX, W, Wg, Ug, bg, h0, c0, lwT, lb).T


def kernel(X, edge_index, edge_weight, W, Wf, Wj, Wc, Wo, Uf, Uj, Uc, Uo,
           bf, bj, bc, bo, h_init, c_init, lin_w, lin_b):
    T, N, F = X.shape
    H = W.shape[1]
    E = edge_index.shape[1]

    chunk_grain = _NS * _CH
    Epad = ((E + chunk_grain - 1) // chunk_grain) * chunk_grain
    pad = Epad - E
    srcp = jnp.concatenate([edge_index[0], jnp.zeros((pad,), jnp.int32)])
    dstp = jnp.concatenate([edge_index[1], jnp.zeros((pad,), jnp.int32)])
    wp = jnp.concatenate(
        [edge_weight, jnp.zeros((T, pad), jnp.float32)], axis=1)

    AX = _sc_spmm(X.reshape(T * N * 2, _HF), srcp, dstp, wp, T, N, F)

    Wg = jnp.concatenate([Wf, Wj, Wo, Wc], axis=1)
    Ug = jnp.concatenate([Uf, Uj, Uo, Uc], axis=1)
    bg = jnp.concatenate([bf, bj, bo, bc]).reshape(1, 4 * H)
    return _lstm(AX, W, Wg, Ug, bg, h_init.reshape(1, H),
                 c_init.reshape(1, H), lin_w.reshape(H, 1),
                 lin_b.reshape(1, 1))


# speedup vs baseline: 1.2390x; 1.2390x over previous
import functools

import jax
import jax.numpy as jnp
from jax import lax
from jax.experimental import pallas as pl
from jax.experimental.pallas import tpu as pltpu
from jax.experimental.pallas import tpu_sc as plsc

_NC = 2
_NS = 16
_L = 16
_CH = 512
_HF = 64
_ZB = 125


def _sc_spmm(Xh, srcp, dstp, wp, T, N, F):
    TT = T // _NC
    Epad = srcp.shape[0]
    ep_tile = Epad // _NS
    nchunk = ep_tile // _CH
    stripe = N // _NS
    nf = _HF // _L

    mesh = plsc.VectorSubcoreMesh(core_axis_name="c", subcore_axis_name="s",
                                  num_cores=_NC, num_subcores=_NS)

    @functools.partial(
        pl.kernel,
        out_type=jax.ShapeDtypeStruct((T, N, F), jnp.float32),
        mesh=mesh,
        compiler_params=pltpu.CompilerParams(use_tc_tiling_on_sc=False),
        scratch_types=[
            pltpu.VMEM((2, _CH), jnp.int32),
            pltpu.VMEM((2, _CH), jnp.int32),
            pltpu.VMEM((2, _CH), jnp.int32),
            pltpu.VMEM((2, _CH), jnp.float32),
            pltpu.VMEM((2, _CH, _HF), jnp.float32),
            pltpu.VMEM((_ZB, _HF), jnp.float32),
            pltpu.VMEM_SHARED((N, _HF), jnp.float32),
            pltpu.SemaphoreType.DMA,
            pltpu.SemaphoreType.DMA,
            pltpu.SemaphoreType.DMA,
            pltpu.SemaphoreType.DMA,
        ],
    )
    def spmm(x_hbm, src_hbm, dst_hbm, w_hbm, out_hbm,
             src_buf, idx_buf, dst_buf, w_buf, rows_buf, zero_buf, acc,
             isem0, isem1, gsem0, gsem1):
        c = lax.axis_index("c")
        s = lax.axis_index("s")
        row0 = s * stripe
        isem = [isem0, isem1]
        gsem = [gsem0, gsem1]

        zvec = jnp.zeros((_L,), jnp.float32)

        def zrow_body(r, _):
            for ff in range(nf):
                zero_buf[r, pl.ds(ff * _L, _L)] = zvec
            return 0
        lax.fori_loop(0, _ZB, zrow_body, 0)

        for tt in range(TT):
            t = c * TT + tt
            for half in range(2):
                base_idx = (t * N) * 2 + half

                def start_in(b, ci):
                    e0 = s * ep_tile + ci * _CH
                    pltpu.async_copy(src_hbm.at[pl.ds(e0, _CH)],
                                     src_buf.at[b], isem[b])
                    pltpu.async_copy(dst_hbm.at[pl.ds(e0, _CH)],
                                     dst_buf.at[b], isem[b])
                    pltpu.async_copy(w_hbm.at[t, pl.ds(e0, _CH)],
                                     w_buf.at[b], isem[b])

                def wait_in(b):
                    pltpu.make_async_copy(src_hbm.at[pl.ds(0, _CH)],
                                          src_buf.at[b], isem[b]).wait()
                    pltpu.make_async_copy(dst_hbm.at[pl.ds(0, _CH)],
                                          dst_buf.at[b], isem[b]).wait()
                    pltpu.make_async_copy(w_hbm.at[0, pl.ds(0, _CH)],
                                          w_buf.at[b], isem[b]).wait()

                def launch_gather(b):
                    def off_body(g, _):
                        sl = pl.ds(g * _L, _L)
                        idx_buf[b, sl] = src_buf[b, sl] * 2 + base_idx
                        return 0
                    lax.fori_loop(0, _CH // _L, off_body, 0)
                    pltpu.async_copy(x_hbm.at[idx_buf.at[b]],
                                     rows_buf.at[b], gsem[b])

                def wait_gather(b):
                    pltpu.make_async_copy(x_hbm.at[pl.ds(0, _CH)],
                                          rows_buf.at[b], gsem[b]).wait()

                def scale_scatter(b):
                    def scale_body(g, _):
                        w16 = w_buf[b, pl.ds(g * _L, _L)]
                        base = g * _L
                        for l in range(_L):
                            wl = lax.broadcast(w16[l], (_L,))
                            for ff in range(nf):
                                fsl = pl.ds(ff * _L, _L)
                                rows_buf[b, base + l, fsl] = (
                                    rows_buf[b, base + l, fsl] * wl)
                        return 0
                    lax.fori_loop(0, _CH // _L, scale_body, 0)
                    pltpu.sync_copy(rows_buf.at[b], acc.at[dst_buf.at[b]],
                                    add=True)

                for z in range(stripe // _ZB):
                    pltpu.sync_copy(
                        zero_buf, acc.at[pl.ds(row0 + z * _ZB, _ZB), :])
                plsc.subcore_barrier()

                start_in(0, 0)
                wait_in(0)
                launch_gather(0)
                start_in(1, 1)

                def pipe_body(g2, _):
                    gi = g2 * 2
                    for b in range(2):
                        ci = gi + b
                        wait_gather(b)
                        scale_scatter(b)
                        start_in(b, jnp.minimum(ci + 2, nchunk - 1))
                        wait_in(1 - b)
                        launch_gather(1 - b)
                    return 0
                lax.fori_loop(0, nchunk // 2, pipe_body, 0)
                wait_gather(0)
                wait_in(1)
                plsc.subcore_barrier()

                pltpu.sync_copy(
                    acc.at[pl.ds(row0, stripe), :],
                    out_hbm.at[t, pl.ds(row0, stripe),
                               pl.ds(half * _HF, _HF)])
                plsc.subcore_barrier()

    return spmm(Xh, srcp, dstp, wp)


def _bdot(a, b):
    return jnp.dot(a.astype(jnp.bfloat16), b.astype(jnp.bfloat16),
                   preferred_element_type=jnp.float32)


def _lstm_body(ax_ref, w_ref, wg_ref, ug_ref, bg_ref, h0_ref, c0_ref,
               lw_ref, lb_ref, out_ref):
    T, B, H = ax_ref.shape[0], ax_ref.shape[1], wg_ref.shape[0]
    w = w_ref[...]
    h = jnp.broadcast_to(h0_ref[...], (B, H))
    c = jnp.broadcast_to(c0_ref[...], (B, H))
    wg = wg_ref[...]
    ug = ug_ref[...]
    bg = bg_ref[...]
    lw = lw_ref[...]
    lb = lb_ref[0, 0]
    zs = []
    for t in range(T):
        y = jnp.maximum(_bdot(ax_ref[t], w), 0.0)
        g = _bdot(y, wg) + _bdot(h, ug) + bg
        f = jax.nn.sigmoid(g[:, 0:H])
        j = jax.nn.sigmoid(g[:, H:2 * H])
        o = jax.nn.sigmoid(g[:, 2 * H:3 * H])
        ct = jax.nn.sigmoid(g[:, 3 * H:4 * H])
        c = j * ct + f * c
        h = o * jnp.tanh(c)
        zs.append(_bdot(h, lw) + lb)
    out_ref[...] = jnp.concatenate(zs, axis=1)


def _lstm(AX, W, Wg, Ug, bg, h0, c0, lwT, lb):
    T, N, F = AX.shape
    H = W.shape[1]
    B = 2000
    return pl.pallas_call(
        _lstm_body,
        grid=(N // B,),
        in_specs=[
            pl.BlockSpec((T, B, F), lambda i: (0, i, 0)),
            pl.BlockSpec((F, H), lambda i: (0, 0)),
            pl.BlockSpec((H, 4 * H), lambda i: (0, 0)),
            pl.BlockSpec((H, 4 * H), lambda i: (0, 0)),
            pl.BlockSpec((1, 4 * H), lambda i: (0, 0)),
            pl.BlockSpec((1, H), lambda i: (0, 0)),
            pl.BlockSpec((1, H), lambda i: (0, 0)),
            pl.BlockSpec((H, 1), lambda i: (0, 0)),
            pl.BlockSpec((1, 1), lambda i: (0, 0)),
        ],
        out_specs=pl.BlockSpec((B, T), lambda i: (i, 0)),
        out_shape=jax.ShapeDtypeStruct((N, T), jnp.float32),
    )(AX, W, Wg, Ug, bg, h0, c0, lwT, lb).T


def kernel(X, edge_index, edge_weight, W, Wf, Wj, Wc, Wo, Uf, Uj, Uc, Uo,
           bf, bj, bc, bo, h_init, c_init, lin_w, lin_b):
    T, N, F = X.shape
    H = W.shape[1]
    E = edge_index.shape[1]

    chunk_grain = _NS * _CH
    Epad = ((E + chunk_grain - 1) // chunk_grain) * chunk_grain
    pad = Epad - E
    srcp = jnp.concatenate([edge_index[0], jnp.zeros((pad,), jnp.int32)])
    dstp = jnp.concatenate([edge_index[1], jnp.zeros((pad,), jnp.int32)])
    wp = jnp.concatenate(
        [edge_weight, jnp.zeros((T, pad), jnp.float32)], axis=1)

    AX = _sc_spmm(X.reshape(T * N * 2, _HF), srcp, dstp, wp, T, N, F)

    Wg = jnp.concatenate([Wf, Wj, Wo, Wc], axis=1)
    Ug = jnp.concatenate([Uf, Uj, Uo, Uc], axis=1)
    bg = jnp.concatenate([bf, bj, bo, bc]).reshape(1, 4 * H)
    return _lstm(AX, W, Wg, Ug, bg, h_init.reshape(1, H),
                 c_init.reshape(1, H), lin_w.reshape(H, 1),
                 lin_b.reshape(1, 1))
